# baseline (speedup 1.0000x reference)
import jax
import jax.numpy as jnp
from jax import lax
from jax.experimental import pallas as pl

B = 8192
N = 64
F = 16
H = 64
C = 128

BL = 256


def _f32dot(a, b, dims):
    return lax.dot_general(a, b, (dims, ((), ())),
                           preferred_element_type=jnp.float32)


def _tc_body(adjT_ref, nobsT_ref, aidT_ref, cent_ref,
             we_ref, be_ref, wms_ref,
             w1c_ref, w1h_ref, b1_ref, w2_ref, b2_ref, wv_ref, bv_ref,
             out_ref):
    f32 = jnp.float32
    aid = aidT_ref[...]
    mask = (lax.broadcasted_iota(jnp.int32, (N, BL), 0) == aid).astype(f32)

    adjT = adjT_ref[...]
    arow = jnp.zeros((N, BL), f32)
    for n1 in range(N):
        arow = arow + mask[n1:n1 + 1, :] * adjT[n1]
    deg = jnp.sum(arow, axis=0, keepdims=True)
    aT = arow / (deg + 1e-6)

    nobsT = nobsT_ref[...]
    we = we_ref[...]
    be = be_ref[...]
    m = jnp.zeros((H, BL), f32)
    nobs_ego = jnp.zeros((F, BL), f32)
    for n in range(N):
        h_n = jax.nn.relu(_f32dot(we, nobsT[n], ((0,), (0,))) + be)
        m = m + aT[n:n + 1, :] * h_n
        nobs_ego = nobs_ego + mask[n:n + 1, :] * nobsT[n]
    ego = jax.nn.relu(_f32dot(we, nobs_ego, ((0,), (0,))) + be)

    p = jnp.concatenate([m, ego], axis=0)
    nbd = jax.nn.relu(_f32dot(p, wms_ref[...], ((0,), (0,))))

    x = jax.nn.relu(
        _f32dot(cent_ref[...], w1c_ref[...], ((1,), (0,)))
        + _f32dot(nbd, w1h_ref[...], ((1,), (0,)))
        + b1_ref[...]
    )
    x = jax.nn.relu(_f32dot(x, w2_ref[...], ((1,), (0,))) + b2_ref[...])
    out_ref[...] = jnp.sum(x * wv_ref[...], axis=1, keepdims=True) + bv_ref[...]


def kernel(cent_obs, node_obs, adj, agent_id, rnn_states, masks,
           W_embed, b_embed, W_msg, W_self, W1, b1, W2, b2, Wv, bv):
    adjT = jnp.transpose(adj, (1, 2, 0))
    nobsT = jnp.transpose(node_obs, (1, 2, 0))
    aidT = agent_id.astype(jnp.int32).reshape(1, B)
    wms = jnp.concatenate([W_msg, W_self], axis=0)

    grid = (B // BL,)
    full = lambda *s: pl.BlockSpec(s, lambda i: (0,) * len(s))
    values = pl.pallas_call(
        _tc_body,
        grid=grid,
        in_specs=[
            pl.BlockSpec((N, N, BL), lambda i: (0, 0, i)),
            pl.BlockSpec((N, F, BL), lambda i: (0, 0, i)),
            pl.BlockSpec((1, BL), lambda i: (0, i)),
            pl.BlockSpec((BL, C), lambda i: (i, 0)),
            full(F, H),
            full(H, 1),
            full(2 * H, H),
            full(C, H),
            full(H, H),
            full(1, H),
            full(H, H),
            full(1, H),
            full(1, H),
            full(1, 1),
        ],
        out_specs=pl.BlockSpec((BL, 1), lambda i: (i, 0)),
        out_shape=jax.ShapeDtypeStruct((B, 1), jnp.float32),
    )(adjT, nobsT, aidT, cent_obs,
      W_embed, b_embed.reshape(H, 1), wms,
      W1[:C], W1[C:], b1.reshape(1, H), W2, b2.reshape(1, H),
      Wv.reshape(1, H), bv.reshape(1, 1))
    return values, rnn_states


# speedup vs baseline: 1.0979x; 1.0979x over previous
import jax
import jax.numpy as jnp
from jax import lax
from jax.experimental import pallas as pl

B = 8192
N = 64
F = 16
H = 64
C = 128

BL = 512


def _f32dot(a, b, dims):
    return lax.dot_general(a, b, (dims, ((), ())),
                           preferred_element_type=jnp.float32)


def _tc_body(adjT_ref, nobsT_ref, aidT_ref, cent_ref,
             we_ref, be_ref, wms_ref,
             w1c_ref, w1h_ref, b1_ref, w2_ref, b2_ref, wv_ref, bv_ref,
             out_ref):
    f32 = jnp.float32
    aid = aidT_ref[...]
    mask = (lax.broadcasted_iota(jnp.int32, (N, BL), 0) == aid).astype(f32)

    adjT = adjT_ref[...]
    arow = jnp.zeros((N, BL), f32)
    for n1 in range(N):
        arow = arow + mask[n1:n1 + 1, :] * adjT[n1]
    deg = jnp.sum(arow, axis=0, keepdims=True)
    aT = arow / (deg + 1e-6)

    nobsT = nobsT_ref[...]
    we = we_ref[...]
    be = be_ref[...]
    m = jnp.zeros((H, BL), f32)
    nobs_ego = jnp.zeros((F, BL), f32)
    for n in range(N):
        h_n = jax.nn.relu(_f32dot(we, nobsT[n], ((0,), (0,))) + be)
        m = m + aT[n:n + 1, :] * h_n
        nobs_ego = nobs_ego + mask[n:n + 1, :] * nobsT[n]
    ego = jax.nn.relu(_f32dot(we, nobs_ego, ((0,), (0,))) + be)

    p = jnp.concatenate([m, ego], axis=0)
    nbd = jax.nn.relu(_f32dot(p, wms_ref[...], ((0,), (0,))))

    x = jax.nn.relu(
        _f32dot(cent_ref[...], w1c_ref[...], ((1,), (0,)))
        + _f32dot(nbd, w1h_ref[...], ((1,), (0,)))
        + b1_ref[...]
    )
    x = jax.nn.relu(_f32dot(x, w2_ref[...], ((1,), (0,))) + b2_ref[...])
    out_ref[...] = jnp.sum(x * wv_ref[...], axis=1, keepdims=True) + bv_ref[...]


def kernel(cent_obs, node_obs, adj, agent_id, rnn_states, masks,
           W_embed, b_embed, W_msg, W_self, W1, b1, W2, b2, Wv, bv):
    adjT = jnp.transpose(adj, (1, 2, 0))
    nobsT = jnp.transpose(node_obs, (1, 2, 0))
    aidT = agent_id.astype(jnp.int32).reshape(1, B)
    wms = jnp.concatenate([W_msg, W_self], axis=0)

    grid = (B // BL,)
    full = lambda *s: pl.BlockSpec(s, lambda i: (0,) * len(s))
    values = pl.pallas_call(
        _tc_body,
        grid=grid,
        in_specs=[
            pl.BlockSpec((N, N, BL), lambda i: (0, 0, i)),
            pl.BlockSpec((N, F, BL), lambda i: (0, 0, i)),
            pl.BlockSpec((1, BL), lambda i: (0, i)),
            pl.BlockSpec((BL, C), lambda i: (i, 0)),
            full(F, H),
            full(H, 1),
            full(2 * H, H),
            full(C, H),
            full(H, H),
            full(1, H),
            full(H, H),
            full(1, H),
            full(1, H),
            full(1, 1),
        ],
        out_specs=pl.BlockSpec((BL, 1), lambda i: (i, 0)),
        out_shape=jax.ShapeDtypeStruct((B, 1), jnp.float32),
    )(adjT, nobsT, aidT, cent_obs,
      W_embed, b_embed.reshape(H, 1), wms,
      W1[:C], W1[C:], b1.reshape(1, H), W2, b2.reshape(1, H),
      Wv.reshape(1, H), bv.reshape(1, 1))
    return values, rnn_states
